# baseline (speedup 1.0000x reference)
import functools
import math

import jax
import jax.numpy as jnp
from jax import lax
from jax.experimental import pallas as pl
from jax.experimental.pallas import tpu as pltpu
from jax.experimental.pallas import tpu_sc as plsc

D_MODEL = 128
SCALE = math.sqrt(float(D_MODEL))
NUM_CORES = 2
NUM_SUBCORES = 16
NW = NUM_CORES * NUM_SUBCORES
CHUNK = 128
LANES = 16
NBUF = 5
PRE = 3


@functools.partial(jax.jit, static_argnames=("n_chunks",))
def _emb_call(idx, lut, n_chunks):
    B = NW * n_chunks * CHUNK
    per_w = n_chunks * CHUNK
    assert n_chunks % NBUF == 0

    mesh = plsc.VectorSubcoreMesh(core_axis_name="c", subcore_axis_name="s")

    @functools.partial(
        pl.kernel,
        out_type=jax.ShapeDtypeStruct((B, D_MODEL), jnp.float32),
        mesh=mesh,
        scratch_types=[
            pltpu.VMEM((per_w,), jnp.int32),
            pltpu.VMEM((NBUF, CHUNK, D_MODEL), jnp.float32),
            pltpu.SemaphoreType.DMA((NBUF,)),
            pltpu.SemaphoreType.DMA((NBUF,)),
        ],
    )
    def emb(idx_hbm, lut_hbm, out_hbm, idx_v, bufs, gsem, ssem):
        wid = lax.axis_index("s") * NUM_CORES + lax.axis_index("c")
        base = wid * per_w
        pltpu.sync_copy(idx_hbm.at[pl.ds(base, per_w)], idx_v)

        def gather_copy(j, b):
            return pltpu.make_async_copy(
                lut_hbm.at[idx_v.at[pl.ds(j * CHUNK, CHUNK)]],
                bufs.at[b],
                gsem.at[b],
            )

        def scatter_copy(j, b):
            return pltpu.make_async_copy(
                bufs.at[b],
                out_hbm.at[pl.ds(base + j * CHUNK, CHUNK)],
                ssem.at[b],
            )

        for b in range(PRE):
            gather_copy(b, b).start()

        def group_body(g, carry):
            for bs in range(NBUF):
                j = g * NBUF + bs
                gather_copy(j, bs).wait()

                def row_body(r, c2):
                    for k in range(D_MODEL // LANES):
                        sl = pl.ds(k * LANES, LANES)
                        bufs[bs, r, sl] = bufs[bs, r, sl] * SCALE
                    return c2

                lax.fori_loop(0, CHUNK, row_body, 0, unroll=8)

                bn = (bs + PRE) % NBUF
                jn = j + PRE

                @pl.when(jn < n_chunks)
                def _():
                    @pl.when(jn >= NBUF)
                    def _():
                        scatter_copy(jn - NBUF, bn).wait()

                    gather_copy(jn, bn).start()

                scatter_copy(j, bs).start()
            return carry

        lax.fori_loop(0, n_chunks // NBUF, group_body, 0)

        for bs in range(NBUF):
            scatter_copy(n_chunks - NBUF + bs, bs).wait()

    return emb(idx, lut)


def kernel(x, lut):
    n_rows, seq = x.shape
    B = n_rows * seq
    n_chunks = B // (NW * CHUNK)
    idx = jnp.transpose(x).reshape(-1).astype(jnp.int32)
    out = _emb_call(idx, lut, n_chunks)
    return jnp.transpose(out.reshape(seq, n_rows, D_MODEL), (1, 0, 2))


# speedup vs baseline: 1.0285x; 1.0285x over previous
import functools
import math

import jax
import jax.numpy as jnp
from jax import lax
from jax.experimental import pallas as pl
from jax.experimental.pallas import tpu as pltpu
from jax.experimental.pallas import tpu_sc as plsc

D_MODEL = 128
SCALE = math.sqrt(float(D_MODEL))
NUM_CORES = 2
NUM_SUBCORES = 16
NW = NUM_CORES * NUM_SUBCORES
CHUNK = 128
LANES = 16
NBUF = 5
PRE = 3


@functools.partial(jax.jit, static_argnames=())
def _emb_call(idx2, lut):
    seq, n_rows = idx2.shape
    n_chunks = seq
    B = seq * n_rows

    mesh = plsc.VectorSubcoreMesh(core_axis_name="c", subcore_axis_name="s")

    @functools.partial(
        pl.kernel,
        out_type=jax.ShapeDtypeStruct((B, D_MODEL), jnp.float32),
        mesh=mesh,
        scratch_types=[
            pltpu.VMEM((seq, CHUNK), jnp.int32),
            pltpu.VMEM((NBUF, CHUNK, D_MODEL), jnp.float32),
            pltpu.SemaphoreType.DMA((NBUF,)),
            pltpu.SemaphoreType.DMA((NBUF,)),
        ],
    )
    def emb(idx_hbm, lut_hbm, out_hbm, idx_v, bufs, gsem, ssem):
        wid = lax.axis_index("s") * NUM_CORES + lax.axis_index("c")
        col0 = wid * CHUNK
        pltpu.sync_copy(idx_hbm.at[:, pl.ds(col0, CHUNK)], idx_v)

        def gather_copy(j, b):
            return pltpu.make_async_copy(
                lut_hbm.at[idx_v.at[j]],
                bufs.at[b],
                gsem.at[b],
            )

        def scatter_copy(j, b):
            return pltpu.make_async_copy(
                bufs.at[b],
                out_hbm.at[pl.ds(j * n_rows + col0, CHUNK)],
                ssem.at[b],
            )

        for b in range(PRE):
            gather_copy(b, b).start()

        def group_body(g, carry):
            for bs in range(NBUF):
                j = g * NBUF + bs
                gather_copy(j, bs).wait()

                def row_body(r, c2):
                    for k in range(D_MODEL // LANES):
                        sl = pl.ds(k * LANES, LANES)
                        bufs[bs, r, sl] = bufs[bs, r, sl] * SCALE
                    return c2

                lax.fori_loop(0, CHUNK, row_body, 0, unroll=8)

                bn = (bs + PRE) % NBUF
                jn = j + PRE

                @pl.when(jn < n_chunks)
                def _():
                    @pl.when(jn >= NBUF)
                    def _():
                        scatter_copy(jn - NBUF, bn).wait()

                    gather_copy(jn, bn).start()

                scatter_copy(j, bs).start()
            return carry

        lax.fori_loop(0, n_chunks // NBUF, group_body, 0)

        for bs in range(NBUF):
            scatter_copy(n_chunks - NBUF + bs, bs).wait()

    return emb(idx2, lut)


def kernel(x, lut):
    n_rows, seq = x.shape
    out = _emb_call(jnp.transpose(x).astype(jnp.int32), lut)
    return jnp.transpose(out.reshape(seq, n_rows, D_MODEL), (1, 0, 2))


# speedup vs baseline: 1.0380x; 1.0093x over previous
import functools
import math

import jax
import jax.numpy as jnp
from jax import lax
from jax.experimental import pallas as pl
from jax.experimental.pallas import tpu as pltpu
from jax.experimental.pallas import tpu_sc as plsc

D_MODEL = 128
SCALE = math.sqrt(float(D_MODEL))
NUM_CORES = 2
NUM_SUBCORES = 16
NW = NUM_CORES * NUM_SUBCORES
CHUNK = 128
LANES = 16
NBUF = 5
PRE = 3


@functools.partial(jax.jit, static_argnames=())
def _emb_call(idx2, lut):
    seq, n_rows = idx2.shape
    n_chunks = seq
    B = seq * n_rows

    mesh = plsc.VectorSubcoreMesh(core_axis_name="c", subcore_axis_name="s")

    @functools.partial(
        pl.kernel,
        out_type=jax.ShapeDtypeStruct((B, D_MODEL), jnp.float32),
        mesh=mesh,
        scratch_types=[
            pltpu.VMEM((seq, CHUNK), jnp.int32),
            pltpu.VMEM((NBUF, CHUNK, D_MODEL), jnp.float32),
            pltpu.SemaphoreType.DMA((NBUF,)),
            pltpu.SemaphoreType.DMA((NBUF,)),
        ],
    )
    def emb(idx_hbm, lut_hbm, out_hbm, idx_v, bufs, gsem, ssem):
        wid = lax.axis_index("s") * NUM_CORES + lax.axis_index("c")
        col0 = wid * CHUNK
        pltpu.sync_copy(idx_hbm.at[:, pl.ds(col0, CHUNK)], idx_v)

        def gather_copy(j, b):
            return pltpu.make_async_copy(
                lut_hbm.at[idx_v.at[j]],
                bufs.at[b],
                gsem.at[b],
            )

        def scatter_copy(j, b):
            return pltpu.make_async_copy(
                bufs.at[b],
                out_hbm.at[pl.ds(j * n_rows + col0, CHUNK)],
                ssem.at[b],
            )

        for b in range(PRE):
            gather_copy(b, b).start()

        def group_body(g, carry):
            for bs in range(NBUF):
                j = g * NBUF + bs
                gather_copy(j, bs).wait()

                def row_body(r, c2):
                    for k in range(D_MODEL // LANES):
                        sl = pl.ds(k * LANES, LANES)
                        bufs[bs, r, sl] = bufs[bs, r, sl] * SCALE
                    return c2

                lax.fori_loop(0, CHUNK, row_body, 0, unroll=2)

                bn = (bs + PRE) % NBUF
                jn = j + PRE

                @pl.when(jn < n_chunks)
                def _():
                    @pl.when(jn >= NBUF)
                    def _():
                        scatter_copy(jn - NBUF, bn).wait()

                    gather_copy(jn, bn).start()

                scatter_copy(j, bs).start()
            return carry

        lax.fori_loop(0, n_chunks // NBUF, group_body, 0)

        for bs in range(NBUF):
            scatter_copy(n_chunks - NBUF + bs, bs).wait()

    return emb(idx2, lut)


def kernel(x, lut):
    n_rows, seq = x.shape
    out = _emb_call(jnp.transpose(x).astype(jnp.int32), lut)
    return jnp.transpose(out.reshape(seq, n_rows, D_MODEL), (1, 0, 2))


# speedup vs baseline: 1.0450x; 1.0067x over previous
import functools
import math

import jax
import jax.numpy as jnp
from jax import lax
from jax.experimental import pallas as pl
from jax.experimental.pallas import tpu as pltpu
from jax.experimental.pallas import tpu_sc as plsc

D_MODEL = 128
SCALE = math.sqrt(float(D_MODEL))
NUM_CORES = 2
NUM_SUBCORES = 16
NW = NUM_CORES * NUM_SUBCORES
COLS = 128
SPLIT = 2
CHUNK = COLS // SPLIT
LANES = 16
NBUF = 10
PRE = 6


@functools.partial(jax.jit, static_argnames=())
def _emb_call(idx2, lut):
    seq, n_rows = idx2.shape
    n_chunks = seq * SPLIT
    B = seq * n_rows
    assert n_chunks % NBUF == 0 and NBUF % SPLIT == 0 and PRE % SPLIT == 0

    mesh = plsc.VectorSubcoreMesh(core_axis_name="c", subcore_axis_name="s")

    @functools.partial(
        pl.kernel,
        out_type=jax.ShapeDtypeStruct((B, D_MODEL), jnp.float32),
        mesh=mesh,
        scratch_types=[
            pltpu.VMEM((seq, COLS), jnp.int32),
            pltpu.VMEM((NBUF, CHUNK, D_MODEL), jnp.float32),
            pltpu.SemaphoreType.DMA((NBUF,)),
            pltpu.SemaphoreType.DMA((NBUF,)),
        ],
    )
    def emb(idx_hbm, lut_hbm, out_hbm, idx_v, bufs, gsem, ssem):
        wid = lax.axis_index("s") * NUM_CORES + lax.axis_index("c")
        col0 = wid * COLS
        pltpu.sync_copy(idx_hbm.at[:, pl.ds(col0, COLS)], idx_v)

        def gather_copy(plane, half, b):
            return pltpu.make_async_copy(
                lut_hbm.at[idx_v.at[plane, pl.ds(half * CHUNK, CHUNK)]],
                bufs.at[b],
                gsem.at[b],
            )

        def scatter_copy(plane, half, b):
            return pltpu.make_async_copy(
                bufs.at[b],
                out_hbm.at[pl.ds(plane * n_rows + col0 + half * CHUNK, CHUNK)],
                ssem.at[b],
            )

        for b in range(PRE):
            gather_copy(b // SPLIT, b % SPLIT, b).start()

        def group_body(g, carry):
            p0 = g * (NBUF // SPLIT)
            for bs in range(NBUF):
                plane, half = p0 + bs // SPLIT, bs % SPLIT
                gather_copy(plane, half, bs).wait()

                def row_body(r, c2):
                    for k in range(D_MODEL // LANES):
                        sl = pl.ds(k * LANES, LANES)
                        bufs[bs, r, sl] = bufs[bs, r, sl] * SCALE
                    return c2

                lax.fori_loop(0, CHUNK, row_body, 0, unroll=2)

                bn = (bs + PRE) % NBUF
                pn, hn = p0 + (bs + PRE) // SPLIT, bs % SPLIT
                jn = (plane * SPLIT + half) + PRE

                @pl.when(jn < n_chunks)
                def _():
                    @pl.when(jn >= NBUF)
                    def _():
                        scatter_copy(pn - NBUF // SPLIT, hn, bn).wait()

                    gather_copy(pn, hn, bn).start()

                scatter_copy(plane, half, bs).start()
            return carry

        lax.fori_loop(0, n_chunks // NBUF, group_body, 0)

        for bs in range(NBUF):
            j = n_chunks - NBUF + bs
            scatter_copy(j // SPLIT, j % SPLIT, bs).wait()

    return emb(idx2, lut)


def kernel(x, lut):
    n_rows, seq = x.shape
    out = _emb_call(jnp.transpose(x).astype(jnp.int32), lut)
    return jnp.transpose(out.reshape(seq, n_rows, D_MODEL), (1, 0, 2))


# speedup vs baseline: 1.0503x; 1.0051x over previous
import functools
import math

import jax
import jax.numpy as jnp
from jax import lax
from jax.experimental import pallas as pl
from jax.experimental.pallas import tpu as pltpu
from jax.experimental.pallas import tpu_sc as plsc

D_MODEL = 128
SCALE = math.sqrt(float(D_MODEL))
NUM_CORES = 2
NUM_SUBCORES = 16
NW = NUM_CORES * NUM_SUBCORES
COLS = 128
SPLIT = 2
CHUNK = COLS // SPLIT
LANES = 16
NBUF = 10
PRE = 8


@functools.partial(jax.jit, static_argnames=())
def _emb_call(idx2, lut):
    seq, n_rows = idx2.shape
    n_chunks = seq * SPLIT
    B = seq * n_rows
    assert n_chunks % NBUF == 0 and NBUF % SPLIT == 0 and PRE % SPLIT == 0

    mesh = plsc.VectorSubcoreMesh(core_axis_name="c", subcore_axis_name="s")

    @functools.partial(
        pl.kernel,
        out_type=jax.ShapeDtypeStruct((B, D_MODEL), jnp.float32),
        mesh=mesh,
        scratch_types=[
            pltpu.VMEM((seq, COLS), jnp.int32),
            pltpu.VMEM((NBUF, CHUNK, D_MODEL), jnp.float32),
            pltpu.SemaphoreType.DMA((NBUF,)),
            pltpu.SemaphoreType.DMA((NBUF,)),
        ],
    )
    def emb(idx_hbm, lut_hbm, out_hbm, idx_v, bufs, gsem, ssem):
        wid = lax.axis_index("s") * NUM_CORES + lax.axis_index("c")
        col0 = wid * COLS
        pltpu.sync_copy(idx_hbm.at[:, pl.ds(col0, COLS)], idx_v)

        def gather_copy(plane, half, b):
            return pltpu.make_async_copy(
                lut_hbm.at[idx_v.at[plane, pl.ds(half * CHUNK, CHUNK)]],
                bufs.at[b],
                gsem.at[b],
            )

        def scatter_copy(plane, half, b):
            return pltpu.make_async_copy(
                bufs.at[b],
                out_hbm.at[pl.ds(plane * n_rows + col0 + half * CHUNK, CHUNK)],
                ssem.at[b],
            )

        for b in range(PRE):
            gather_copy(b // SPLIT, b % SPLIT, b).start()

        def group_body(g, carry):
            p0 = g * (NBUF // SPLIT)
            for bs in range(NBUF):
                plane, half = p0 + bs // SPLIT, bs % SPLIT
                gather_copy(plane, half, bs).wait()

                def row_body(r, c2):
                    for k in range(D_MODEL // LANES):
                        sl = pl.ds(k * LANES, LANES)
                        bufs[bs, r, sl] = bufs[bs, r, sl] * SCALE
                    return c2

                lax.fori_loop(0, CHUNK, row_body, 0, unroll=2)

                bn = (bs + PRE) % NBUF
                pn, hn = p0 + (bs + PRE) // SPLIT, bs % SPLIT
                jn = (plane * SPLIT + half) + PRE

                @pl.when(jn < n_chunks)
                def _():
                    @pl.when(jn >= NBUF)
                    def _():
                        scatter_copy(pn - NBUF // SPLIT, hn, bn).wait()

                    gather_copy(pn, hn, bn).start()

                scatter_copy(plane, half, bs).start()
            return carry

        lax.fori_loop(0, n_chunks // NBUF, group_body, 0)

        for bs in range(NBUF):
            j = n_chunks - NBUF + bs
            scatter_copy(j // SPLIT, j % SPLIT, bs).wait()

    return emb(idx2, lut)


def kernel(x, lut):
    n_rows, seq = x.shape
    out = _emb_call(jnp.transpose(x).astype(jnp.int32), lut)
    return jnp.transpose(out.reshape(seq, n_rows, D_MODEL), (1, 0, 2))
